# Initial kernel scaffold; baseline (speedup 1.0000x reference)
import jax
import jax.numpy as jnp
from jax.experimental import pallas as pl


def kernel(t, x, embed, wq, bq, wk, bk):
    raise NotImplementedError("write your pallas kernel here")


# speedup vs baseline: 33.7647x; 33.7647x over previous
import jax
import jax.numpy as jnp
from jax.experimental import pallas as pl

_SCALE = 0.1767766952966369


def _body(x_ref, e_ref, wq_ref, bq_ref, wk_ref, bk_ref, y_ref):
    x = x_ref[...]
    e = e_ref[...]
    m = jnp.dot(x, e, preferred_element_type=jnp.float32)
    vsum = jnp.sum(x, axis=1, keepdims=True)
    s = jnp.dot(m, wq_ref[...].T, preferred_element_type=jnp.float32) + vsum * bq_ref[...]
    u = jnp.dot(s, wk_ref[...], preferred_element_type=jnp.float32)
    c = jnp.dot(s, bk_ref[...].T, preferred_element_type=jnp.float32)
    fx = (jnp.dot(u, e.T, preferred_element_type=jnp.float32) + c) * _SCALE
    y_ref[...] = jnp.where(x != 0.0, x * (1.0 + fx), 0.0)


def kernel(t, x, embed, wq, bq, wk, bk):
    del t
    e = embed[1:]
    return pl.pallas_call(
        _body,
        out_shape=jax.ShapeDtypeStruct(x.shape, x.dtype),
    )(x, e, wq, bq.reshape(1, -1), wk, bk.reshape(1, -1))
